# baseline (speedup 1.0000x reference)
import functools

import jax
import jax.numpy as jnp
from jax import lax
from jax.experimental import pallas as pl
from jax.experimental.pallas import tpu as pltpu
from jax.experimental.pallas import tpu_sc as plsc

_NC = 2
_NS = 16
_NW = _NC * _NS
_BC = 16


def _make_sc(n, s, d):
    b_per_w = n // _NW
    n_chunks = b_per_w // _BC
    rows_c = _BC * s
    mesh = plsc.VectorSubcoreMesh(core_axis_name="c", subcore_axis_name="s")

    @functools.partial(
        pl.kernel,
        mesh=mesh,
        out_type=jax.ShapeDtypeStruct((n, s, 2), jnp.float32),
        scratch_types=[
            pltpu.VMEM((rows_c, 8), jnp.float32),
            pltpu.VMEM((rows_c, 8), jnp.float32),
            pltpu.VMEM((_BC, s, 2), jnp.float32),
            pltpu.VMEM((_BC, s, 2), jnp.float32),
            pltpu.SemaphoreType.DMA,
            pltpu.SemaphoreType.DMA,
            pltpu.SemaphoreType.DMA,
            pltpu.SemaphoreType.DMA,
        ],
        compiler_params=pltpu.CompilerParams(
            use_tc_tiling_on_sc=False, needs_layout_passes=False
        ),
    )
    def _sc(x_hbm, out_hbm, vb0, vb1, cb0, cb1, si0, si1, so0, so1):
        wid = lax.axis_index("s") * _NC + lax.axis_index("c")
        b_base = wid * b_per_w
        lane = lax.iota(jnp.int32, 16)
        vbs, cbs, sis, sos = (vb0, vb1), (cb0, cb1), (si0, si1), (so0, so1)

        def in_copy(c):
            b0 = b_base + c * _BC
            return pltpu.make_async_copy(
                x_hbm.at[pl.ds(b0 * s, rows_c), 0:8], vbs[c % 2], sis[c % 2]
            )

        def out_copy(c):
            b0 = b_base + c * _BC
            return pltpu.make_async_copy(
                cbs[c % 2], out_hbm.at[pl.ds(b0, _BC), :, :], sos[c % 2]
            )

        in_copy(0).start()
        for c in range(n_chunks):
            if c + 1 < n_chunks:
                in_copy(c + 1).start()
            in_copy(c).wait()
            if c >= 2:
                out_copy(c - 2).wait()
            vb, cb = vbs[c % 2], cbs[c % 2]

            col = lane & 1

            def pack_outer(bh, _):
                bh_vec = jnp.full((16,), bh, jnp.int32)
                row0 = bh * s

                def pack_inner(m, s_idx):
                    vals = plsc.load_gather(vb, [row0 + s_idx, col])
                    plsc.store_scatter(cb, [bh_vec, s_idx, col], vals)
                    return s_idx + 8

                lax.fori_loop(0, 2 * s // 16, pack_inner, lane >> 1,
                              unroll=5)
                return _

            lax.fori_loop(0, _BC, pack_outer, None)
            out_copy(c).start()
        out_copy(n_chunks - 2).wait()
        out_copy(n_chunks - 1).wait()

    return _sc


def kernel(x):
    n, s, d = x.shape
    return _make_sc(n, s, d)(x.reshape(n * s, d))


# speedup vs baseline: 1.0108x; 1.0108x over previous
import functools

import jax
import jax.numpy as jnp
from jax import lax
from jax.experimental import pallas as pl
from jax.experimental.pallas import tpu as pltpu
from jax.experimental.pallas import tpu_sc as plsc

_NC = 2
_NS = 16
_NW = _NC * _NS
_BC = 16


def _make_sc(n, s, d):
    b_per_w = n // _NW
    n_chunks = b_per_w // _BC
    rows_c = _BC * s
    mesh = plsc.VectorSubcoreMesh(core_axis_name="c", subcore_axis_name="s")

    @functools.partial(
        pl.kernel,
        mesh=mesh,
        out_type=jax.ShapeDtypeStruct((n, s, 2), jnp.float32),
        scratch_types=[
            pltpu.VMEM((rows_c, 8), jnp.float32),
            pltpu.VMEM((rows_c, 8), jnp.float32),
            pltpu.VMEM((_BC, s, 2), jnp.float32),
            pltpu.VMEM((_BC, s, 2), jnp.float32),
            pltpu.SemaphoreType.DMA,
            pltpu.SemaphoreType.DMA,
            pltpu.SemaphoreType.DMA,
            pltpu.SemaphoreType.DMA,
        ],
        compiler_params=pltpu.CompilerParams(
            use_tc_tiling_on_sc=False, needs_layout_passes=False
        ),
    )
    def _sc(x_hbm, out_hbm, vb0, vb1, cb0, cb1, si0, si1, so0, so1):
        wid = lax.axis_index("s") * _NC + lax.axis_index("c")
        b_base = wid * b_per_w
        lane = lax.iota(jnp.int32, 16)
        vbs, cbs, sis, sos = (vb0, vb1), (cb0, cb1), (si0, si1), (so0, so1)

        def in_copy(c):
            b0 = b_base + c * _BC
            return pltpu.make_async_copy(
                x_hbm.at[pl.ds(b0 * s, rows_c), 0:8], vbs[c % 2], sis[c % 2]
            )

        def out_copy(c):
            b0 = b_base + c * _BC
            return pltpu.make_async_copy(
                cbs[c % 2], out_hbm.at[pl.ds(b0, _BC), :, :], sos[c % 2]
            )

        in_copy(0).start()
        for c in range(n_chunks):
            if c + 1 < n_chunks:
                in_copy(c + 1).start()
            in_copy(c).wait()
            if c >= 2:
                out_copy(c - 2).wait()
            vb, cb = vbs[c % 2], cbs[c % 2]

            def pack_outer(bh, _):
                bh_vec = jnp.full((16,), bh, jnp.int32)

                def pack_inner(m, _):
                    k = m * 16 + lane
                    row = bh * s + (k >> 1)
                    col = k & 1
                    vals = plsc.load_gather(vb, [row, col])
                    plsc.store_scatter(cb, [bh_vec, k >> 1, col], vals)
                    return _

                lax.fori_loop(0, 2 * s // 16, pack_inner, None)
                return _

            lax.fori_loop(0, _BC, pack_outer, None)
            out_copy(c).start()
        out_copy(n_chunks - 2).wait()
        out_copy(n_chunks - 1).wait()

    return _sc


def kernel(x):
    n, s, d = x.shape
    return _make_sc(n, s, d)(x.reshape(n * s, d))
